# baseline (speedup 1.0000x reference)
import functools

import jax
import jax.numpy as jnp
from jax import lax
from jax.experimental import pallas as pl
from jax.experimental.pallas import tpu as pltpu
from jax.experimental.pallas import tpu_sc as plsc


def _vq_body(z_ref, w_ref, wsq_ref, idx_ref, loss_ref):
    e = z_ref[0]
    w = w_ref[...]
    wsq = wsq_ref[0]
    zsq = jnp.sum(e * e, axis=0)
    mm = jax.lax.dot_general(w, e, (((1,), (0,)), ((), ())),
                            preferred_element_type=jnp.float32)
    s = (zsq[None, :] + wsq[:, None]) - 2.0 * mm
    smin = jnp.min(s, axis=0)
    kio = jax.lax.broadcasted_iota(jnp.int32, s.shape, 0)
    amin = jnp.min(jnp.where(s == smin[None, :], kio, jnp.int32(1 << 30)),
                   axis=0)
    idx_ref[0] = amin[None, :]
    tot = jnp.sum(smin)
    loss_ref[0] = jnp.full((1, 128), 1.25 * tot / 65536.0, jnp.float32)


def _tc_argmin(z_e, W):
    B, C, H, Wd = z_e.shape
    P = H * Wd
    K = W.shape[0]
    z = z_e.reshape(B, C, P)
    wsq = jnp.sum(W ** 2, axis=-1)[None, :]
    idx, loss = pl.pallas_call(
        _vq_body,
        grid=(B,),
        in_specs=[
            pl.BlockSpec((1, C, P), lambda b: (b, 0, 0)),
            pl.BlockSpec((K, C), lambda b: (0, 0)),
            pl.BlockSpec((1, K), lambda b: (0, 0)),
        ],
        out_specs=[
            pl.BlockSpec((1, 1, P), lambda b: (b, 0, 0)),
            pl.BlockSpec((1, 1, 128), lambda b: (b, 0, 0)),
        ],
        out_shape=[
            jax.ShapeDtypeStruct((B, 1, P), jnp.int32),
            jax.ShapeDtypeStruct((B, 1, 128), jnp.float32),
        ],
    )(z, W, wsq)
    return idx.reshape(B * P), loss[:, 0, 0]


def _sc_gather(W, idx):
    info = plsc.get_sparse_core_info()
    nw = info.num_cores * info.num_subcores
    B = idx.shape[0]
    D = W.shape[1]
    b_per_w = B // nw
    mesh = plsc.VectorSubcoreMesh(core_axis_name="c", subcore_axis_name="s")

    @functools.partial(
        pl.kernel, mesh=mesh,
        out_type=jax.ShapeDtypeStruct((B, D), jnp.float32),
        scratch_types=[
            pltpu.VMEM((b_per_w,), jnp.int32),
            pltpu.VMEM((b_per_w, D), jnp.float32),
            pltpu.SemaphoreType.DMA,
        ],
    )
    def k(table_hbm, idx_hbm, out_hbm, idx_v, rows_v, sem):
        wid = lax.axis_index("s") * info.num_cores + lax.axis_index("c")
        base = wid * b_per_w
        pltpu.sync_copy(idx_hbm.at[pl.ds(base, b_per_w)], idx_v)
        pltpu.async_copy(table_hbm.at[idx_v], rows_v, sem).wait()
        pltpu.sync_copy(rows_v, out_hbm.at[pl.ds(base, b_per_w)])

    return k(W, idx)


def kernel(z_e, W):
    B, C, H, Wd = z_e.shape
    idx, loss = _tc_argmin(z_e, W)
    Wp = jnp.pad(W, ((0, 0), (0, 128 - W.shape[1])))
    rows = _sc_gather(Wp, idx)
    out = jnp.transpose(rows.reshape(B, H, Wd, 128)[..., :C], (0, 3, 1, 2))
    return out, loss


# speedup vs baseline: 1.4139x; 1.4139x over previous
import jax
import jax.numpy as jnp
from jax.experimental import pallas as pl


def _vq_body(z_ref, w_ref, wsq_ref, out_ref, loss_ref):
    e = z_ref[0]
    w = w_ref[...]
    wsq = wsq_ref[0]
    zsq = jnp.sum(e * e, axis=0)
    mm = jax.lax.dot_general(w, e, (((1,), (0,)), ((), ())),
                            preferred_element_type=jnp.float32)
    s = (zsq[None, :] + wsq[:, None]) - 2.0 * mm
    smin = jnp.min(s, axis=0)
    kio = jax.lax.broadcasted_iota(jnp.int32, s.shape, 0)
    amin = jnp.min(jnp.where(s == smin[None, :], kio, jnp.int32(1 << 30)),
                   axis=0)
    onehot = (kio == amin[None, :]).astype(jnp.float32)
    zq = jax.lax.dot_general(w, onehot, (((0,), (0,)), ((), ())),
                             preferred_element_type=jnp.float32)
    out_ref[0] = zq
    tot = jnp.sum(smin)
    loss_ref[0] = jnp.full((1, 128), 1.25 * tot / 65536.0, jnp.float32)


def kernel(z_e, W):
    B, C, H, Wd = z_e.shape
    P = H * Wd
    K = W.shape[0]
    z = z_e.reshape(B, C, P)
    wsq = jnp.sum(W ** 2, axis=-1)[None, :]
    out, loss = pl.pallas_call(
        _vq_body,
        grid=(B,),
        in_specs=[
            pl.BlockSpec((1, C, P), lambda b: (b, 0, 0)),
            pl.BlockSpec((K, C), lambda b: (0, 0)),
            pl.BlockSpec((1, K), lambda b: (0, 0)),
        ],
        out_specs=[
            pl.BlockSpec((1, C, P), lambda b: (b, 0, 0)),
            pl.BlockSpec((1, 1, 128), lambda b: (b, 0, 0)),
        ],
        out_shape=[
            jax.ShapeDtypeStruct((B, C, P), jnp.float32),
            jax.ShapeDtypeStruct((B, 1, 128), jnp.float32),
        ],
    )(z, W, wsq)
    return out.reshape(B, C, H, Wd), loss[:, 0, 0]
